# baseline (speedup 1.0000x reference)
import functools

import jax
import jax.numpy as jnp
from jax import lax
from jax.experimental import pallas as pl
from jax.experimental.pallas import tpu as pltpu
from jax.experimental.pallas import tpu_sc as plsc

B, S, D, E = 4, 8192, 768, 8
N = B * S
T = 2048
CHUNKS = 16
NC = N // CHUNKS

NEG_INF = float("-inf")


def _route_tc(h, e_iota):
    m1 = jnp.max(h, axis=1, keepdims=True)
    i1 = jnp.min(jnp.where(h == m1, e_iota, E), axis=1, keepdims=True)
    h2 = jnp.where(e_iota == i1, NEG_INF, h)
    m2 = jnp.max(h2, axis=1, keepdims=True)
    i2 = jnp.min(jnp.where(h2 == m2, e_iota, E), axis=1, keepdims=True)
    w2 = jnp.exp(m2 - m1)
    recip = 1.0 / (1.0 + w2)
    return jnp.where(e_iota == i1, recip,
                     jnp.where(e_iota == i2, w2 * recip, 0.0))


def _logits_tile(x_ref, wg_ref, wn_ref, bg_ref, bn_ref, eps_ref):
    x = x_ref[...]
    gate = lax.dot_general(
        x, wg_ref[...], (((1,), (0,)), ((), ())),
        preferred_element_type=jnp.float32) + bg_ref[...]
    noise = lax.dot_general(
        x, wn_ref[...], (((1,), (0,)), ((), ())),
        preferred_element_type=jnp.float32) + bn_ref[...]
    return gate + eps_ref[...] * jax.nn.softplus(noise)


def _logits_body(x_ref, wg_ref, wn_ref, bg_ref, bn_ref, eps_ref, out_ref):
    out_ref[...] = _logits_tile(x_ref, wg_ref, wn_ref, bg_ref, bn_ref,
                                eps_ref)


def _fused_body(x_ref, wg_ref, wn_ref, bg_ref, bn_ref, eps_ref, out_ref):
    h = _logits_tile(x_ref, wg_ref, wn_ref, bg_ref, bn_ref, eps_ref)
    e_iota = lax.broadcasted_iota(jnp.int32, h.shape, 1)
    out_ref[...] = _route_tc(h, e_iota)


def _tc_call(body, x2, wg_t, wn_t, bg2, bn2, eps2, tile, row0, rows,
             out_rows=None, out_row0=0):
    if out_rows is None:
        out_rows = rows
    ib = row0 // tile
    ob = out_row0 // tile
    return pl.pallas_call(
        body,
        grid=(rows // tile,),
        in_specs=[
            pl.BlockSpec((tile, D), lambda i: (ib + i, 0)),
            pl.BlockSpec((D, E), lambda i: (0, 0)),
            pl.BlockSpec((D, E), lambda i: (0, 0)),
            pl.BlockSpec((1, E), lambda i: (0, 0)),
            pl.BlockSpec((1, E), lambda i: (0, 0)),
            pl.BlockSpec((tile, E), lambda i: (ib + i, 0)),
        ],
        out_specs=pl.BlockSpec((tile, E), lambda i: (ob + i, 0)),
        out_shape=jax.ShapeDtypeStruct((out_rows, E), jnp.float32),
        compiler_params=pltpu.CompilerParams(
            dimension_semantics=("arbitrary",)),
    )(x2, wg_t, wn_t, bg2, bn2, eps2)


_INFO = plsc.get_sparse_core_info()
_NW = _INFO.num_cores * _INFO.num_subcores
_L = _INFO.num_lanes
_TOK = NC // _NW
_GROUPS = _TOK // _L


def _route_body(h_hbm, out_hbm, h_v, out_v):
    wid = lax.axis_index("s") * _INFO.num_cores + lax.axis_index("c")
    tok0 = wid * _TOK
    pltpu.sync_copy(h_hbm.at[pl.ds(tok0, _TOK), :], h_v)
    lane = lax.iota(jnp.int32, _L)
    cols = [jnp.full((_L,), e, jnp.int32) for e in range(E)]

    def group(g, carry):
        rows = lane + g * _L
        v = [plsc.load_gather(h_v, [rows, cols[e]]) for e in range(E)]
        best = v[0]
        besti = jnp.zeros((_L,), jnp.int32)
        for e in range(1, E):
            gt = v[e] > best
            best = jnp.where(gt, v[e], best)
            besti = jnp.where(gt, e, besti)
        second = jnp.full((_L,), NEG_INF, jnp.float32)
        secondi = jnp.full((_L,), E, jnp.int32)
        for e in range(E):
            gt = jnp.logical_and(v[e] > second, besti != e)
            second = jnp.where(gt, v[e], second)
            secondi = jnp.where(gt, e, secondi)
        w2 = jnp.exp(second - best)
        w1n = 1.0 / (1.0 + w2)
        w2n = w2 * w1n
        zero = jnp.zeros((_L,), jnp.float32)
        for e in range(E):
            out_e = jnp.where(besti == e, w1n,
                              jnp.where(secondi == e, w2n, zero))
            plsc.store_scatter(out_v, [rows, cols[e]], out_e)
        return carry

    lax.fori_loop(0, _GROUPS, group, 0)
    pltpu.sync_copy(out_v, out_hbm.at[pl.ds(tok0, _TOK), :])


_route_sc = functools.partial(
    pl.kernel,
    out_type=jax.ShapeDtypeStruct((NC, E), jnp.float32),
    mesh=plsc.VectorSubcoreMesh(core_axis_name="c", subcore_axis_name="s"),
    scratch_types=[
        pltpu.VMEM((_TOK, E), jnp.float32),
        pltpu.VMEM((_TOK, E), jnp.float32),
    ],
    compiler_params=pltpu.CompilerParams(needs_layout_passes=False),
)(_route_body)


@jax.jit
def _gating(x2, wg_t, wn_t, bg2, bn2, eps2):
    h0 = _tc_call(_logits_body, x2, wg_t, wn_t, bg2, bn2, eps2,
                  tile=512, row0=0, rows=NC)
    g0 = _route_sc(h0)
    g_full = _tc_call(_fused_body, x2, wg_t, wn_t, bg2, bn2, eps2,
                      tile=T, row0=NC, rows=N - NC,
                      out_rows=N, out_row0=NC)
    return lax.dynamic_update_slice(g_full, g0, (0, 0))


def kernel(x, Wg, bg, Wnoise, bnoise, eps):
    g = _gating(x.reshape(N, D), Wg.T, Wnoise.T, bg.reshape(1, E),
                bnoise.reshape(1, E), eps.reshape(N, E))
    return g.reshape(B, S, E)


# speedup vs baseline: 1.0235x; 1.0235x over previous
import functools

import jax
import jax.numpy as jnp
from jax import lax
from jax.experimental import pallas as pl
from jax.experimental.pallas import tpu as pltpu
from jax.experimental.pallas import tpu_sc as plsc

B, S, D, E = 4, 8192, 768, 8
N = B * S
T = 2048
CHUNKS = 16
NC = N // CHUNKS

NEG_INF = float("-inf")


def _route_tc(h, e_iota):
    m1 = jnp.max(h, axis=1, keepdims=True)
    i1 = jnp.min(jnp.where(h == m1, e_iota, E), axis=1, keepdims=True)
    h2 = jnp.where(e_iota == i1, NEG_INF, h)
    m2 = jnp.max(h2, axis=1, keepdims=True)
    i2 = jnp.min(jnp.where(h2 == m2, e_iota, E), axis=1, keepdims=True)
    w2 = jnp.exp(m2 - m1)
    recip = 1.0 / (1.0 + w2)
    return jnp.where(e_iota == i1, recip,
                     jnp.where(e_iota == i2, w2 * recip, 0.0))


def _logits_tile(x_ref, wg_ref, wn_ref, bg_ref, bn_ref, eps_ref):
    x = x_ref[...]
    gate = lax.dot_general(
        x, wg_ref[...], (((1,), (0,)), ((), ())),
        preferred_element_type=jnp.float32) + bg_ref[...]
    noise = lax.dot_general(
        x, wn_ref[...], (((1,), (0,)), ((), ())),
        preferred_element_type=jnp.float32) + bn_ref[...]
    return gate + eps_ref[...] * jax.nn.softplus(noise)


def _logits_body(x_ref, wg_ref, wn_ref, bg_ref, bn_ref, eps_ref, out_ref):
    out_ref[...] = _logits_tile(x_ref, wg_ref, wn_ref, bg_ref, bn_ref,
                                eps_ref)


def _fused_body(x_ref, wg_ref, wn_ref, bg_ref, bn_ref, eps_ref, out_ref):
    h = _logits_tile(x_ref, wg_ref, wn_ref, bg_ref, bn_ref, eps_ref)
    e_iota = lax.broadcasted_iota(jnp.int32, h.shape, 1)
    out_ref[...] = _route_tc(h, e_iota)


def _tc_call(body, x2, wg_t, wn_t, bg2, bn2, eps2, tile, row0, rows,
             out_rows=None, out_row0=0):
    if out_rows is None:
        out_rows = rows
    ib = row0 // tile
    ob = out_row0 // tile
    return pl.pallas_call(
        body,
        grid=(rows // tile,),
        in_specs=[
            pl.BlockSpec((tile, D), lambda i: (ib + i, 0)),
            pl.BlockSpec((D, E), lambda i: (0, 0)),
            pl.BlockSpec((D, E), lambda i: (0, 0)),
            pl.BlockSpec((1, E), lambda i: (0, 0)),
            pl.BlockSpec((1, E), lambda i: (0, 0)),
            pl.BlockSpec((tile, E), lambda i: (ib + i, 0)),
        ],
        out_specs=pl.BlockSpec((tile, E), lambda i: (ob + i, 0)),
        out_shape=jax.ShapeDtypeStruct((out_rows, E), jnp.float32),
        compiler_params=pltpu.CompilerParams(
            dimension_semantics=("arbitrary",)),
    )(x2, wg_t, wn_t, bg2, bn2, eps2)


_INFO = plsc.get_sparse_core_info()
_NW = _INFO.num_cores * _INFO.num_subcores
_L = _INFO.num_lanes
_TOK = NC // _NW
_GROUPS = _TOK // _L


def _route_body(h_hbm, out_hbm, h_v, out_v):
    wid = lax.axis_index("s") * _INFO.num_cores + lax.axis_index("c")
    tok0 = wid * _TOK
    pltpu.sync_copy(h_hbm.at[pl.ds(tok0, _TOK), :], h_v)
    lane = lax.iota(jnp.int32, _L)
    cols = [jnp.full((_L,), e, jnp.int32) for e in range(E)]

    def group(g, carry):
        rows = lane + g * _L
        v = [plsc.load_gather(h_v, [rows, cols[e]]) for e in range(E)]
        best = v[0]
        besti = jnp.zeros((_L,), jnp.int32)
        for e in range(1, E):
            gt = v[e] > best
            best = jnp.where(gt, v[e], best)
            besti = jnp.where(gt, e, besti)
        second = jnp.full((_L,), NEG_INF, jnp.float32)
        secondi = jnp.full((_L,), E, jnp.int32)
        for e in range(E):
            gt = jnp.logical_and(v[e] > second, besti != e)
            second = jnp.where(gt, v[e], second)
            secondi = jnp.where(gt, e, secondi)
        w2 = jnp.exp(second - best)
        w1n = 1.0 / (1.0 + w2)
        w2n = w2 * w1n
        zero = jnp.zeros((_L,), jnp.float32)
        for e in range(E):
            out_e = jnp.where(besti == e, w1n,
                              jnp.where(secondi == e, w2n, zero))
            plsc.store_scatter(out_v, [rows, cols[e]], out_e)
        return carry

    lax.fori_loop(0, _GROUPS, group, 0)
    pltpu.sync_copy(out_v, out_hbm.at[pl.ds(tok0, _TOK), :])


_route_sc = functools.partial(
    pl.kernel,
    out_type=jax.ShapeDtypeStruct((NC, E), jnp.float32),
    mesh=plsc.VectorSubcoreMesh(core_axis_name="c", subcore_axis_name="s"),
    scratch_types=[
        pltpu.VMEM((_TOK, E), jnp.float32),
        pltpu.VMEM((_TOK, E), jnp.float32),
    ],
    compiler_params=pltpu.CompilerParams(needs_layout_passes=False),
)(_route_body)


@jax.jit
def _gating(x2, wg_t, wn_t, bg2, bn2, eps2):
    h0 = _tc_call(_logits_body, x2, wg_t, wn_t, bg2, bn2, eps2,
                  tile=512, row0=0, rows=NC)
    g0 = _route_sc(h0)
    g_rest = _tc_call(_fused_body, x2, wg_t, wn_t, bg2, bn2, eps2,
                      tile=T, row0=NC, rows=N - NC)
    return jnp.concatenate([g0, g_rest], axis=0)


def kernel(x, Wg, bg, Wnoise, bnoise, eps):
    g = _gating(x.reshape(N, D), Wg.T, Wnoise.T, bg.reshape(1, E),
                bnoise.reshape(1, E), eps.reshape(N, E))
    return g.reshape(B, S, E)


# speedup vs baseline: 1.0388x; 1.0149x over previous
import functools

import jax
import jax.numpy as jnp
from jax import lax
from jax.experimental import pallas as pl
from jax.experimental.pallas import tpu as pltpu
from jax.experimental.pallas import tpu_sc as plsc

B, S, D, E = 4, 8192, 768, 8
N = B * S
T = 2048
CHUNKS = 16
NC = N // CHUNKS

NEG_INF = float("-inf")


def _route_tc(h, e_iota):
    m1 = jnp.max(h, axis=1, keepdims=True)
    i1 = jnp.min(jnp.where(h == m1, e_iota, E), axis=1, keepdims=True)
    h2 = jnp.where(e_iota == i1, NEG_INF, h)
    m2 = jnp.max(h2, axis=1, keepdims=True)
    i2 = jnp.min(jnp.where(h2 == m2, e_iota, E), axis=1, keepdims=True)
    w2 = jnp.exp(m2 - m1)
    recip = 1.0 / (1.0 + w2)
    return jnp.where(e_iota == i1, recip,
                     jnp.where(e_iota == i2, w2 * recip, 0.0))


def _logits_tile(x_ref, wg_ref, wn_ref, bg_ref, bn_ref, eps_ref):
    x = x_ref[...]
    gate = lax.dot_general(
        x, wg_ref[...], (((1,), (0,)), ((), ())),
        preferred_element_type=jnp.float32) + bg_ref[...]
    noise = lax.dot_general(
        x, wn_ref[...], (((1,), (0,)), ((), ())),
        preferred_element_type=jnp.float32) + bn_ref[...]
    return gate + eps_ref[...] * jax.nn.softplus(noise)


def _logits_body(x_ref, wg_ref, wn_ref, bg_ref, bn_ref, eps_ref, out_ref):
    out_ref[...] = _logits_tile(x_ref, wg_ref, wn_ref, bg_ref, bn_ref,
                                eps_ref)


def _fused_body(x_ref, wg_ref, wn_ref, bg_ref, bn_ref, eps_ref, out_ref):
    h = _logits_tile(x_ref, wg_ref, wn_ref, bg_ref, bn_ref, eps_ref)
    e_iota = lax.broadcasted_iota(jnp.int32, h.shape, 1)
    out_ref[...] = _route_tc(h, e_iota)


def _tc_call(body, x2, wg_t, wn_t, bg2, bn2, eps2, tile, row0, rows,
             out_rows=None, out_row0=0):
    if out_rows is None:
        out_rows = rows
    ib = row0 // tile
    ob = out_row0 // tile
    return pl.pallas_call(
        body,
        grid=(rows // tile,),
        in_specs=[
            pl.BlockSpec((tile, D), lambda i: (ib + i, 0)),
            pl.BlockSpec((D, E), lambda i: (0, 0)),
            pl.BlockSpec((D, E), lambda i: (0, 0)),
            pl.BlockSpec((1, E), lambda i: (0, 0)),
            pl.BlockSpec((1, E), lambda i: (0, 0)),
            pl.BlockSpec((tile, E), lambda i: (ib + i, 0)),
        ],
        out_specs=pl.BlockSpec((tile, E), lambda i: (ob + i, 0)),
        out_shape=jax.ShapeDtypeStruct((out_rows, E), jnp.float32),
        compiler_params=pltpu.CompilerParams(
            dimension_semantics=("arbitrary",)),
    )(x2, wg_t, wn_t, bg2, bn2, eps2)


_INFO = plsc.get_sparse_core_info()
_NW = _INFO.num_subcores
_L = _INFO.num_lanes
_TOK = NC // _NW
_GROUPS = _TOK // _L


def _route_body(h_hbm, out_hbm, h_v, out_v):
    wid = lax.axis_index("s")
    tok0 = wid * _TOK
    pltpu.sync_copy(h_hbm.at[pl.ds(tok0, _TOK), :], h_v)
    lane = lax.iota(jnp.int32, _L)
    cols = [jnp.full((_L,), e, jnp.int32) for e in range(E)]

    def group(g, carry):
        rows = lane + g * _L
        v = [plsc.load_gather(h_v, [rows, cols[e]]) for e in range(E)]
        best = v[0]
        besti = jnp.zeros((_L,), jnp.int32)
        for e in range(1, E):
            gt = v[e] > best
            best = jnp.where(gt, v[e], best)
            besti = jnp.where(gt, e, besti)
        second = jnp.full((_L,), NEG_INF, jnp.float32)
        secondi = jnp.full((_L,), E, jnp.int32)
        for e in range(E):
            gt = jnp.logical_and(v[e] > second, besti != e)
            second = jnp.where(gt, v[e], second)
            secondi = jnp.where(gt, e, secondi)
        w2 = jnp.exp(second - best)
        w1n = 1.0 / (1.0 + w2)
        w2n = w2 * w1n
        zero = jnp.zeros((_L,), jnp.float32)
        for e in range(E):
            out_e = jnp.where(besti == e, w1n,
                              jnp.where(secondi == e, w2n, zero))
            plsc.store_scatter(out_v, [rows, cols[e]], out_e)
        return carry

    lax.fori_loop(0, _GROUPS, group, 0)
    pltpu.sync_copy(out_v, out_hbm.at[pl.ds(tok0, _TOK), :])


_route_sc = functools.partial(
    pl.kernel,
    out_type=jax.ShapeDtypeStruct((NC, E), jnp.float32),
    mesh=plsc.VectorSubcoreMesh(core_axis_name="c", subcore_axis_name="s", num_cores=1),
    scratch_types=[
        pltpu.VMEM((_TOK, E), jnp.float32),
        pltpu.VMEM((_TOK, E), jnp.float32),
    ],
    compiler_params=pltpu.CompilerParams(needs_layout_passes=False),
)(_route_body)


@jax.jit
def _gating(x2, wg_t, wn_t, bg2, bn2, eps2):
    h0 = _tc_call(_logits_body, x2, wg_t, wn_t, bg2, bn2, eps2,
                  tile=512, row0=0, rows=NC)
    g0 = _route_sc(h0)
    g_rest = _tc_call(_fused_body, x2, wg_t, wn_t, bg2, bn2, eps2,
                      tile=T, row0=NC, rows=N - NC)
    return jnp.concatenate([g0, g_rest], axis=0)


def kernel(x, Wg, bg, Wnoise, bnoise, eps):
    g = _gating(x.reshape(N, D), Wg.T, Wnoise.T, bg.reshape(1, E),
                bnoise.reshape(1, E), eps.reshape(N, E))
    return g.reshape(B, S, E)
